# baseline (speedup 1.0000x reference)
import functools

import jax
import jax.numpy as jnp
from jax import lax
from jax.experimental import pallas as pl
from jax.experimental.pallas import tpu as pltpu
from jax.experimental.pallas import tpu_sc as plsc

B, L, D = 4096, 50, 128
LP = 128
LANES = 16
DV = D // LANES
NC, NS = 2, 16
NW = NC * NS
S_SC = 2560
RPW = S_SC // NW
K = 256
NIDX = RPW * L + K + 16
TRASH = RPW
RB = 128


def _sc_body(x_hbm, m_hbm, out_hbm, mbuf, idxbuf, tgtbuf, dstbufs, accbuf,
             cntbuf, sems_g):
    wid = lax.axis_index("s") * NC + lax.axis_index("c")
    base = wid * RPW

    pltpu.sync_copy(m_hbm.at[pl.ds(base, RPW)], mbuf)

    zero = jnp.zeros((LANES,), jnp.float32)

    def zero_row(r, c):
        for d in range(DV):
            accbuf[r, pl.ds(d * LANES, LANES)] = zero
        return c

    lax.fori_loop(0, RPW + 1, zero_row, 0)

    lane_iota = lax.iota(jnp.int32, LANES)

    def gen_row(r, off):
        cntv = jnp.zeros((LANES,), jnp.int32)
        brow = base + r
        rvec = jnp.zeros((LANES,), jnp.int32) + r
        for k in range(4):
            mrow = mbuf[r, pl.ds(k * LANES, LANES)]
            mb = mrow > 0.0
            keys = jnp.where(mb, lane_iota, lane_iota + LANES)
            flat = (lane_iota + (k * LANES)) * B + brow
            _, vs = plsc.sort_key_val(keys, flat)
            idxbuf[pl.ds(off, LANES)] = vs
            tgtbuf[pl.ds(off, LANES)] = rvec
            pc = plsc.all_reduce_population_count(mb)
            off = off + pc[0]
            cntv = cntv + pc
        cntbuf[r] = cntv.astype(jnp.float32)
        return off

    v_end = lax.fori_loop(0, RPW, gen_row, jnp.int32(0))

    v_pad = ((v_end + (K - 1)) // K) * K
    safe = jnp.zeros((LANES,), jnp.int32) + base
    trash = jnp.zeros((LANES,), jnp.int32) + TRASH

    def pad_body(off):
        idxbuf[pl.ds(off, LANES)] = safe
        tgtbuf[pl.ds(off, LANES)] = trash
        return off + LANES

    lax.while_loop(lambda o: o < v_pad, pad_body, v_end)
    nch = v_pad // K

    def start_g(i, slot):
        idx = idxbuf.at[pl.ds(i * K, K)]
        return pltpu.async_copy(x_hbm.at[idx], dstbufs[slot], sems_g[slot])

    def wait_g(slot):
        idx = idxbuf.at[pl.ds(0, K)]
        pltpu.make_async_copy(x_hbm.at[idx], dstbufs[slot], sems_g[slot]).wait()

    def accum(i, slot):
        db = dstbufs[slot]

        def jg_body(jg, c):
            tv = tgtbuf[pl.ds(i * K + jg * LANES, LANES)]
            for jj in range(LANES):
                t = tv[jj]
                row = jg * LANES + jj
                for d in range(DV):
                    plsc.addupdate(
                        accbuf.at[t, pl.ds(d * LANES, LANES)],
                        db[row, pl.ds(d * LANES, LANES)],
                    )
            return c

        lax.fori_loop(0, K // LANES, jg_body, 0)

    @pl.when(nch > 0)
    def _():
        start_g(0, 0)

    @pl.when(nch > 1)
    def _():
        start_g(1, 1)

    def pair_body(p, c):
        for b2 in range(2):
            i = p * 2 + b2

            @pl.when(i < nch)
            def _():
                wait_g(b2)
                accum(i, b2)

                @pl.when(i + 2 < nch)
                def _():
                    start_g(i + 2, b2)

        return c

    lax.fori_loop(0, (nch + 1) // 2, pair_body, 0)

    def fin_row(r, c):
        cv = cntbuf[r]
        for d in range(DV):
            sl = pl.ds(d * LANES, LANES)
            accbuf[r, sl] = accbuf[r, sl] / cv
        return c

    lax.fori_loop(0, RPW, fin_row, 0)
    pltpu.sync_copy(accbuf.at[pl.ds(0, RPW)], out_hbm.at[pl.ds(base, RPW)])


def _build_call():
    mesh = plsc.VectorSubcoreMesh(core_axis_name="c", subcore_axis_name="s")
    scratch = (
        pltpu.VMEM((RPW, LP), jnp.float32),
        pltpu.VMEM((NIDX,), jnp.int32),
        pltpu.VMEM((NIDX,), jnp.int32),
        [pltpu.VMEM((K, D), jnp.float32) for _ in range(2)],
        pltpu.VMEM((RPW + 8, D), jnp.float32),
        pltpu.VMEM((RPW, LANES), jnp.float32),
        [pltpu.SemaphoreType.DMA for _ in range(2)],
    )
    return pl.kernel(
        _sc_body,
        out_type=jax.ShapeDtypeStruct((S_SC, D), jnp.float32),
        mesh=mesh,
        scratch_types=scratch,
        compiler_params=pltpu.CompilerParams(needs_layout_passes=False),
    )


_sc_call = _build_call()


def _tc_body(x_ref, m_ref, o_ref):
    acc = x_ref[0] * m_ref[:, 0:1]
    for l in range(1, L):
        acc = acc + x_ref[l] * m_ref[:, l : l + 1]
    cnt = jnp.sum(m_ref[:, :L], axis=1, keepdims=True)
    o_ref[...] = acc / cnt


def _build_tc_call():
    nblk = (B - S_SC) // RB
    off = S_SC // RB
    return pl.pallas_call(
        _tc_body,
        grid=(nblk,),
        in_specs=[
            pl.BlockSpec((L, RB, D), lambda i: (0, off + i, 0)),
            pl.BlockSpec((RB, LP), lambda i: (off + i, 0)),
        ],
        out_specs=pl.BlockSpec((RB, D), lambda i: (i, 0)),
        out_shape=jax.ShapeDtypeStruct((B - S_SC, D), jnp.float32),
    )


_tc_call = _build_tc_call()


@jax.jit
def kernel(inputs, mask):
    maskf = jnp.pad(mask.astype(jnp.float32), ((0, 0), (0, LP - L)))
    xt = jnp.transpose(inputs, (1, 0, 2))
    xflat = jnp.reshape(xt, (L * B, D))
    out_sc = _sc_call(xflat, maskf)
    out_tc = _tc_call(xt, maskf)
    return jnp.concatenate([out_sc, out_tc], axis=0)


# speedup vs baseline: 1.5798x; 1.5798x over previous
import functools

import jax
import jax.numpy as jnp
from jax import lax
from jax.experimental import pallas as pl
from jax.experimental.pallas import tpu as pltpu
from jax.experimental.pallas import tpu_sc as plsc

B, L, D = 4096, 50, 128
LP = 128
LANES = 16
DV = D // LANES
NC, NS = 2, 16
NW = NC * NS
S_SC = 2560
RPW = S_SC // NW
K = 256
NIDX = RPW * L + K + 16
TRASH = RPW
RB = 128


def _sc_body(x_hbm, m_hbm, out_hbm, mbuf, idxbuf, tgtbuf, dstbufs, accbuf,
             cntbuf, shacc, sems_g):
    wid = lax.axis_index("s") * NC + lax.axis_index("c")
    base = wid * RPW
    slot_base = lax.axis_index("s") * (RPW + 8)

    pltpu.sync_copy(m_hbm.at[pl.ds(base, RPW)], mbuf)

    zero = jnp.zeros((LANES,), jnp.float32)

    def zero_row(r, c):
        for d in range(DV):
            accbuf[r, pl.ds(d * LANES, LANES)] = zero
        return c

    lax.fori_loop(0, RPW + 8, zero_row, 0)
    pltpu.sync_copy(accbuf, shacc.at[pl.ds(slot_base, RPW + 8)])

    lane_iota = lax.iota(jnp.int32, LANES)

    def gen_row(r, off):
        cntv = jnp.zeros((LANES,), jnp.int32)
        brow = base + r
        rvec = jnp.zeros((LANES,), jnp.int32) + (slot_base + r)
        for k in range(4):
            mrow = mbuf[r, pl.ds(k * LANES, LANES)]
            mb = mrow > 0.0
            keys = jnp.where(mb, lane_iota, lane_iota + LANES)
            flat = (lane_iota + (k * LANES)) * B + brow
            _, vs = plsc.sort_key_val(keys, flat)
            idxbuf[pl.ds(off, LANES)] = vs
            tgtbuf[pl.ds(off, LANES)] = rvec
            pc = plsc.all_reduce_population_count(mb)
            off = off + pc[0]
            cntv = cntv + pc
        cntbuf[r] = cntv.astype(jnp.float32)
        return off

    v_end = lax.fori_loop(0, RPW, gen_row, jnp.int32(0))

    v_pad = ((v_end + (K - 1)) // K) * K
    safe = jnp.zeros((LANES,), jnp.int32) + base
    trash = jnp.zeros((LANES,), jnp.int32) + (slot_base + TRASH)

    def pad_body(off):
        idxbuf[pl.ds(off, LANES)] = safe
        tgtbuf[pl.ds(off, LANES)] = trash
        return off + LANES

    lax.while_loop(lambda o: o < v_pad, pad_body, v_end)
    nch = v_pad // K

    def start_g(i, slot):
        idx = idxbuf.at[pl.ds(i * K, K)]
        return pltpu.async_copy(x_hbm.at[idx], dstbufs[slot], sems_g[slot])

    def wait_g(slot):
        idx = idxbuf.at[pl.ds(0, K)]
        pltpu.make_async_copy(x_hbm.at[idx], dstbufs[slot], sems_g[slot]).wait()

    def accum(i, slot):
        tgt = tgtbuf.at[pl.ds(i * K, K)]
        pltpu.sync_copy(dstbufs[slot], shacc.at[tgt], add=True)

    @pl.when(nch > 0)
    def _():
        start_g(0, 0)

    @pl.when(nch > 1)
    def _():
        start_g(1, 1)

    def pair_body(p, c):
        for b2 in range(2):
            i = p * 2 + b2

            @pl.when(i < nch)
            def _():
                wait_g(b2)
                accum(i, b2)

                @pl.when(i + 2 < nch)
                def _():
                    start_g(i + 2, b2)

        return c

    lax.fori_loop(0, (nch + 1) // 2, pair_body, 0)

    pltpu.sync_copy(shacc.at[pl.ds(slot_base, RPW)], accbuf.at[pl.ds(0, RPW)])

    def fin_row(r, c):
        cv = cntbuf[r]
        for d in range(DV):
            sl = pl.ds(d * LANES, LANES)
            accbuf[r, sl] = accbuf[r, sl] / cv
        return c

    lax.fori_loop(0, RPW, fin_row, 0)
    pltpu.sync_copy(accbuf.at[pl.ds(0, RPW)], out_hbm.at[pl.ds(base, RPW)])


def _build_call():
    mesh = plsc.VectorSubcoreMesh(core_axis_name="c", subcore_axis_name="s")
    scratch = (
        pltpu.VMEM((RPW, LP), jnp.float32),
        pltpu.VMEM((NIDX,), jnp.int32),
        pltpu.VMEM((NIDX,), jnp.int32),
        [pltpu.VMEM((K, D), jnp.float32) for _ in range(2)],
        pltpu.VMEM((RPW + 8, D), jnp.float32),
        pltpu.VMEM((RPW, LANES), jnp.float32),
        pltpu.VMEM_SHARED((NS * (RPW + 8), D), jnp.float32),
        [pltpu.SemaphoreType.DMA for _ in range(2)],
    )
    return pl.kernel(
        _sc_body,
        out_type=jax.ShapeDtypeStruct((S_SC, D), jnp.float32),
        mesh=mesh,
        scratch_types=scratch,
        compiler_params=pltpu.CompilerParams(needs_layout_passes=False),
    )


_sc_call = _build_call()


def _tc_body(x_ref, m_ref, o_ref):
    acc = x_ref[0] * m_ref[:, 0:1]
    for l in range(1, L):
        acc = acc + x_ref[l] * m_ref[:, l : l + 1]
    cnt = jnp.sum(m_ref[:, :L], axis=1, keepdims=True)
    o_ref[...] = acc / cnt


def _build_tc_call():
    nblk = (B - S_SC) // RB
    off = S_SC // RB
    return pl.pallas_call(
        _tc_body,
        grid=(nblk,),
        in_specs=[
            pl.BlockSpec((L, RB, D), lambda i: (0, off + i, 0)),
            pl.BlockSpec((RB, LP), lambda i: (off + i, 0)),
        ],
        out_specs=pl.BlockSpec((RB, D), lambda i: (i, 0)),
        out_shape=jax.ShapeDtypeStruct((B - S_SC, D), jnp.float32),
    )


_tc_call = _build_tc_call()


@jax.jit
def kernel(inputs, mask):
    maskf = jnp.pad(mask.astype(jnp.float32), ((0, 0), (0, LP - L)))
    xt = jnp.transpose(inputs, (1, 0, 2))
    xflat = jnp.reshape(xt, (L * B, D))
    out_sc = _sc_call(xflat, maskf)
    out_tc = _tc_call(xt, maskf)
    return jnp.concatenate([out_sc, out_tc], axis=0)


# speedup vs baseline: 1.6343x; 1.0345x over previous
import functools

import jax
import jax.numpy as jnp
from jax import lax
from jax.experimental import pallas as pl
from jax.experimental.pallas import tpu as pltpu
from jax.experimental.pallas import tpu_sc as plsc

B, L, D = 4096, 50, 128
LP = 128
LANES = 16
DV = D // LANES
NC, NS = 2, 16
NW = NC * NS
S_SC = 1792
RPW = S_SC // NW
K = 256
NIDX = RPW * L + K + 16
TRASH = RPW
RB = 128


def _sc_body(x_hbm, m_hbm, out_hbm, mbuf, idxbuf, tgtbuf, dstbufs, accbuf,
             cntbuf, shacc, sems_g):
    wid = lax.axis_index("s") * NC + lax.axis_index("c")
    base = wid * RPW
    slot_base = lax.axis_index("s") * (RPW + 8)

    pltpu.sync_copy(m_hbm.at[pl.ds(base, RPW)], mbuf)

    zero = jnp.zeros((LANES,), jnp.float32)

    def zero_row(r, c):
        for d in range(DV):
            accbuf[r, pl.ds(d * LANES, LANES)] = zero
        return c

    lax.fori_loop(0, RPW + 8, zero_row, 0)
    pltpu.sync_copy(accbuf, shacc.at[pl.ds(slot_base, RPW + 8)])

    lane_iota = lax.iota(jnp.int32, LANES)

    def gen_row(r, off):
        cntv = jnp.zeros((LANES,), jnp.int32)
        brow = base + r
        rvec = jnp.zeros((LANES,), jnp.int32) + (slot_base + r)
        for k in range(4):
            mrow = mbuf[r, pl.ds(k * LANES, LANES)]
            mb = mrow > 0.0
            keys = jnp.where(mb, lane_iota, lane_iota + LANES)
            flat = (lane_iota + (k * LANES)) * B + brow
            _, vs = plsc.sort_key_val(keys, flat)
            idxbuf[pl.ds(off, LANES)] = vs
            tgtbuf[pl.ds(off, LANES)] = rvec
            pc = plsc.all_reduce_population_count(mb)
            off = off + pc[0]
            cntv = cntv + pc
        cntbuf[r] = cntv.astype(jnp.float32)
        return off

    v_end = lax.fori_loop(0, RPW, gen_row, jnp.int32(0))

    v_pad = ((v_end + (K - 1)) // K) * K
    safe = jnp.zeros((LANES,), jnp.int32) + base
    trash = jnp.zeros((LANES,), jnp.int32) + (slot_base + TRASH)

    def pad_body(off):
        idxbuf[pl.ds(off, LANES)] = safe
        tgtbuf[pl.ds(off, LANES)] = trash
        return off + LANES

    lax.while_loop(lambda o: o < v_pad, pad_body, v_end)
    nch = v_pad // K

    def start_g(i, slot):
        idx = idxbuf.at[pl.ds(i * K, K)]
        return pltpu.async_copy(x_hbm.at[idx], dstbufs[slot], sems_g[slot])

    def wait_g(slot):
        idx = idxbuf.at[pl.ds(0, K)]
        pltpu.make_async_copy(x_hbm.at[idx], dstbufs[slot], sems_g[slot]).wait()

    def accum(i, slot):
        tgt = tgtbuf.at[pl.ds(i * K, K)]
        pltpu.sync_copy(dstbufs[slot], shacc.at[tgt], add=True)

    @pl.when(nch > 0)
    def _():
        start_g(0, 0)

    @pl.when(nch > 1)
    def _():
        start_g(1, 1)

    def pair_body(p, c):
        for b2 in range(2):
            i = p * 2 + b2

            @pl.when(i < nch)
            def _():
                wait_g(b2)
                accum(i, b2)

                @pl.when(i + 2 < nch)
                def _():
                    start_g(i + 2, b2)

        return c

    lax.fori_loop(0, (nch + 1) // 2, pair_body, 0)

    pltpu.sync_copy(shacc.at[pl.ds(slot_base, RPW)], accbuf.at[pl.ds(0, RPW)])

    def fin_row(r, c):
        cv = cntbuf[r]
        for d in range(DV):
            sl = pl.ds(d * LANES, LANES)
            accbuf[r, sl] = accbuf[r, sl] / cv
        return c

    lax.fori_loop(0, RPW, fin_row, 0)
    pltpu.sync_copy(accbuf.at[pl.ds(0, RPW)], out_hbm.at[pl.ds(base, RPW)])


def _build_call():
    mesh = plsc.VectorSubcoreMesh(core_axis_name="c", subcore_axis_name="s")
    scratch = (
        pltpu.VMEM((RPW, LP), jnp.float32),
        pltpu.VMEM((NIDX,), jnp.int32),
        pltpu.VMEM((NIDX,), jnp.int32),
        [pltpu.VMEM((K, D), jnp.float32) for _ in range(2)],
        pltpu.VMEM((RPW + 8, D), jnp.float32),
        pltpu.VMEM((RPW, LANES), jnp.float32),
        pltpu.VMEM_SHARED((NS * (RPW + 8), D), jnp.float32),
        [pltpu.SemaphoreType.DMA for _ in range(2)],
    )
    return pl.kernel(
        _sc_body,
        out_type=jax.ShapeDtypeStruct((S_SC, D), jnp.float32),
        mesh=mesh,
        scratch_types=scratch,
        compiler_params=pltpu.CompilerParams(needs_layout_passes=False),
    )


_sc_call = _build_call()


def _tc_body(x_ref, m_ref, o_ref):
    acc = x_ref[0] * m_ref[:, 0:1]
    for l in range(1, L):
        acc = acc + x_ref[l] * m_ref[:, l : l + 1]
    cnt = jnp.sum(m_ref[:, :L], axis=1, keepdims=True)
    o_ref[...] = acc / cnt


def _build_tc_call():
    nblk = (B - S_SC) // RB
    off = S_SC // RB
    return pl.pallas_call(
        _tc_body,
        grid=(nblk,),
        in_specs=[
            pl.BlockSpec((L, RB, D), lambda i: (0, off + i, 0)),
            pl.BlockSpec((RB, LP), lambda i: (off + i, 0)),
        ],
        out_specs=pl.BlockSpec((RB, D), lambda i: (i, 0)),
        out_shape=jax.ShapeDtypeStruct((B - S_SC, D), jnp.float32),
    )


_tc_call = _build_tc_call()


@jax.jit
def kernel(inputs, mask):
    maskf = jnp.pad(mask.astype(jnp.float32), ((0, 0), (0, LP - L)))
    xt = jnp.transpose(inputs, (1, 0, 2))
    xflat = jnp.reshape(xt, (L * B, D))
    out_sc = _sc_call(xflat, maskf)
    out_tc = _tc_call(xt, maskf)
    return jnp.concatenate([out_sc, out_tc], axis=0)


# speedup vs baseline: 1.8263x; 1.1174x over previous
import functools

import jax
import jax.numpy as jnp
from jax import lax
from jax.experimental import pallas as pl
from jax.experimental.pallas import tpu as pltpu
from jax.experimental.pallas import tpu_sc as plsc

B, L, D = 4096, 50, 128
LP = 128
LANES = 16
DV = D // LANES
NC, NS = 2, 16
NW = NC * NS
S_SC = 2048
RPW = S_SC // NW
CH = 8
NCHUNK = RPW // CH
NBUF = 2
RB = 128


def _sc_body(x_hbm, m_hbm, out_hbm, xbufs, mbufs, obufs, sems_in, sems_out):
    wid = lax.axis_index("s") * NC + lax.axis_index("c")
    base = wid * RPW

    def start_in(g, slot):
        rows = base + g * CH
        cx = pltpu.async_copy(
            x_hbm.at[:, pl.ds(rows, CH), :], xbufs[slot], sems_in[slot]
        )
        cm = pltpu.async_copy(m_hbm.at[pl.ds(rows, CH)], mbufs[slot], sems_in[slot])
        return (cx, cm)

    def compute(slot):
        xb, mb, ob = xbufs[slot], mbufs[slot], obufs[slot]

        nfull = L // LANES
        ntail = L - nfull * LANES

        def row_body(r, carry):
            del carry

            def grp_body(k, carry):
                accs = list(carry[:DV])
                cnt = carry[DV]
                mrow = mb[r, pl.ds(k * LANES, LANES)]
                for j in range(LANES):
                    m = mrow[j]
                    cnt = cnt + m
                    lpos = k * LANES + j
                    for d in range(DV):
                        accs[d] = accs[d] + xb[lpos, r, pl.ds(d * LANES, LANES)] * m
                return (*accs, cnt)

            init = tuple(jnp.zeros((LANES,), jnp.float32) for _ in range(DV + 1))
            res = lax.fori_loop(0, nfull, grp_body, init)
            accs = list(res[:DV])
            cnt = res[DV]
            mrow = mb[r, pl.ds(nfull * LANES, LANES)]
            for j in range(ntail):
                m = mrow[j]
                cnt = cnt + m
                for d in range(DV):
                    accs[d] = accs[d] + xb[nfull * LANES + j, r, pl.ds(d * LANES, LANES)] * m
            for d in range(DV):
                ob[r, pl.ds(d * LANES, LANES)] = accs[d] / cnt
            return 0

        lax.fori_loop(0, CH, row_body, 0)

    def start_out(g, slot):
        rows = base + g * CH
        return pltpu.async_copy(obufs[slot], out_hbm.at[pl.ds(rows, CH)], sems_out[slot])

    def wait_in(slot):
        pltpu.make_async_copy(
            x_hbm.at[:, pl.ds(0, CH), :], xbufs[slot], sems_in[slot]
        ).wait()
        pltpu.make_async_copy(m_hbm.at[pl.ds(0, CH)], mbufs[slot], sems_in[slot]).wait()

    def wait_out(slot):
        pltpu.make_async_copy(obufs[slot], out_hbm.at[pl.ds(0, CH)], sems_out[slot]).wait()

    for g in range(NBUF):
        start_in(g, g)

    def group_body(gg, carry):
        for b in range(NBUF):
            g = gg * NBUF + b
            wait_in(b)

            @pl.when(g >= NBUF)
            def _():
                wait_out(b)

            compute(b)
            start_out(g, b)

            @pl.when(g + NBUF < NCHUNK)
            def _():
                start_in(g + NBUF, b)

        return carry

    lax.fori_loop(0, NCHUNK // NBUF, group_body, 0)
    for b in range(NBUF):
        wait_out(b)


def _build_call():
    mesh = plsc.VectorSubcoreMesh(core_axis_name="c", subcore_axis_name="s")
    scratch = (
        [pltpu.VMEM((L, CH, D), jnp.float32) for _ in range(NBUF)],
        [pltpu.VMEM((CH, LP), jnp.float32) for _ in range(NBUF)],
        [pltpu.VMEM((CH, D), jnp.float32) for _ in range(NBUF)],
        [pltpu.SemaphoreType.DMA for _ in range(NBUF)],
        [pltpu.SemaphoreType.DMA for _ in range(NBUF)],
    )
    return pl.kernel(
        _sc_body,
        out_type=jax.ShapeDtypeStruct((S_SC, D), jnp.float32),
        mesh=mesh,
        scratch_types=scratch,
    )


_sc_call = _build_call()


def _tc_body(x_ref, m_ref, o_ref):
    acc = x_ref[0] * m_ref[:, 0:1]
    for l in range(1, L):
        acc = acc + x_ref[l] * m_ref[:, l : l + 1]
    cnt = jnp.sum(m_ref[:, :L], axis=1, keepdims=True)
    o_ref[...] = acc / cnt


def _build_tc_call():
    nblk = (B - S_SC) // RB
    off = S_SC // RB
    return pl.pallas_call(
        _tc_body,
        grid=(nblk,),
        in_specs=[
            pl.BlockSpec((L, RB, D), lambda i: (0, off + i, 0)),
            pl.BlockSpec((RB, LP), lambda i: (off + i, 0)),
        ],
        out_specs=pl.BlockSpec((RB, D), lambda i: (i, 0)),
        out_shape=jax.ShapeDtypeStruct((B - S_SC, D), jnp.float32),
    )


_tc_call = _build_tc_call()


@jax.jit
def kernel(inputs, mask):
    maskf = jnp.pad(mask.astype(jnp.float32), ((0, 0), (0, LP - L)))
    xt = jnp.transpose(inputs, (1, 0, 2))
    out_sc = _sc_call(xt, maskf)
    out_tc = _tc_call(xt, maskf)
    return jnp.concatenate([out_sc, out_tc], axis=0)
